# Initial kernel scaffold; baseline (speedup 1.0000x reference)
import jax
import jax.numpy as jnp
from jax.experimental import pallas as pl


def kernel(x, edge_index, edge_attr, batch, lin1_w, lin1_b, root1_w, bias1, lin2_w, lin2_b, root2_w, bias2):
    raise NotImplementedError("write your pallas kernel here")


# speedup vs baseline: 8.2707x; 8.2707x over previous
import functools

import jax
import jax.numpy as jnp
from jax import lax
from jax.experimental import pallas as pl
from jax.experimental.pallas import tpu as pltpu
from jax.experimental.pallas import tpu_sc as plsc

N = 10000
E = 160000
DF = 128
H = 16
C = 10
G = 64

NC = 2
NS = 16
NW = NC * NS

CHUNK = 128
E_PAD = 163840
N_CHUNKS = E_PAD // CHUNK
CH_PER_TILE = N_CHUNKS // NW
N_ACC = 10240
ROWS_PER_TILE = N_ACC // NS
DUMMY_DST = N

_sc_mesh = plsc.VectorSubcoreMesh(
    core_axis_name="c", subcore_axis_name="s", num_cores=NC, num_subcores=NS
)


def _sc_aggregate(table, src2d, dst2d, ea2d):
  @functools.partial(
      pl.kernel,
      out_type=jax.ShapeDtypeStruct((NC, N_ACC, 32), jnp.float32),
      mesh=_sc_mesh,
      compiler_params=pltpu.CompilerParams(
          needs_layout_passes=False, use_tc_tiling_on_sc=False),
      scratch_types=[
          pltpu.VMEM((CH_PER_TILE, CHUNK), jnp.int32),
          pltpu.VMEM((CH_PER_TILE, CHUNK), jnp.int32),
          pltpu.VMEM((CH_PER_TILE * CHUNK,), jnp.float32),
          pltpu.VMEM((CHUNK, 32), jnp.float32),
          pltpu.VMEM((CHUNK, 32), jnp.float32),
          pltpu.VMEM_SHARED((N_ACC, 32), jnp.float32),
          pltpu.SemaphoreType.DMA,
      ],
  )
  def sc_kernel(table_h, src_h, dst_h, ea_h, out_h,
                src_v, dst_v, ea_v, pq_v, m_v, acc, sem):
    cid = lax.axis_index("c")
    sid = lax.axis_index("s")
    wid = sid * NC + cid

    zeros16 = jnp.zeros((16,), jnp.float32)
    ones16 = jnp.ones((16,), jnp.float32)

    def zfill(j, carry):
      m_v[j, 0:16] = zeros16
      m_v[j, 16:32] = zeros16
      return carry

    lax.fori_loop(0, CHUNK, zfill, 0)

    for z in range(ROWS_PER_TILE // CHUNK):
      pltpu.sync_copy(m_v, acc.at[pl.ds(sid * ROWS_PER_TILE + z * CHUNK, CHUNK)])

    def ofill(j, carry):
      m_v[j, 16:32] = ones16
      return carry

    lax.fori_loop(0, CHUNK, ofill, 0)

    base = wid * CH_PER_TILE
    pltpu.sync_copy(src_h.at[pl.ds(base, CH_PER_TILE)], src_v)
    pltpu.sync_copy(dst_h.at[pl.ds(base, CH_PER_TILE)], dst_v)
    pltpu.sync_copy(ea_h.at[pl.ds(base * CHUNK, CH_PER_TILE * CHUNK)], ea_v)

    plsc.subcore_barrier()

    def chunk_body(g, carry):
      pltpu.async_copy(table_h.at[src_v.at[g]], pq_v, sem).wait()
      gbase = g * CHUNK

      def edge_body(j, carry2):
        ea_b = plsc.load_gather(ea_v, [jnp.full((16,), gbase + j, jnp.int32)])
        p = pq_v[j, 0:16]
        q = pq_v[j, 16:32]
        m_v[j, 0:16] = ea_b * p + q
        return carry2

      lax.fori_loop(0, CHUNK, edge_body, 0)
      pltpu.sync_copy(m_v, acc.at[dst_v.at[g]], add=True)
      return carry

    lax.fori_loop(0, CH_PER_TILE, chunk_body, 0)

    plsc.subcore_barrier()

    pltpu.sync_copy(
        acc.at[pl.ds(sid * ROWS_PER_TILE, ROWS_PER_TILE)],
        out_h.at[cid, pl.ds(sid * ROWS_PER_TILE, ROWS_PER_TILE)],
    )

  return sc_kernel(table, src2d, dst2d, ea2d)


def _tc_tables_in(x, w_pq, w_r):
  def body(x_ref, wpq_ref, wr_ref, pq_ref, r_ref):
    xb = x_ref[...]
    pq_ref[...] = jnp.dot(xb, wpq_ref[...], preferred_element_type=jnp.float32)
    r_ref[...] = jnp.dot(xb, wr_ref[...], preferred_element_type=jnp.float32)

  return pl.pallas_call(
      body,
      out_shape=(
          jax.ShapeDtypeStruct((N, 32), jnp.float32),
          jax.ShapeDtypeStruct((N, 16), jnp.float32),
      ),
  )(x, w_pq, w_r)


def _tc_mid(acc, r1, b1, w_pq, w_r):
  def body(acc_ref, r1_ref, b1_ref, wpq_ref, wr_ref, pq2_ref, r2_ref):
    s = acc_ref[0, :N, 0:16] + acc_ref[1, :N, 0:16]
    cnt = acc_ref[0, :N, 16:17] + acc_ref[1, :N, 16:17]
    agg = s / jnp.maximum(cnt, 1.0)
    h1 = jnp.maximum(r1_ref[...] + agg + b1_ref[...], 0.0)
    pq2_ref[...] = jnp.dot(h1, wpq_ref[...], preferred_element_type=jnp.float32)
    r2_ref[...] = jnp.dot(h1, wr_ref[...], preferred_element_type=jnp.float32)

  return pl.pallas_call(
      body,
      out_shape=(
          jax.ShapeDtypeStruct((N, 32), jnp.float32),
          jax.ShapeDtypeStruct((N, 16), jnp.float32),
      ),
  )(acc, r1, b1, w_pq, w_r)


def _tc_pool(acc, r2, b2, batch2d):
  def body(acc_ref, r2_ref, b2_ref, batch_ref, out_ref):
    s = acc_ref[0, :N, 0:16] + acc_ref[1, :N, 0:16]
    cnt = acc_ref[0, :N, 16:17] + acc_ref[1, :N, 16:17]
    h2 = r2_ref[...] + s / jnp.maximum(cnt, 1.0) + b2_ref[...]
    gid = lax.broadcasted_iota(jnp.int32, (1, G), 1)
    oh = (batch_ref[...] == gid).astype(jnp.float32)
    sums = lax.dot_general(oh, h2, (((0,), (0,)), ((), ())),
                           preferred_element_type=jnp.float32)
    gcnt = lax.dot_general(oh, jnp.ones((N, 1), jnp.float32),
                           (((0,), (0,)), ((), ())),
                           preferred_element_type=jnp.float32)
    pooled = sums / jnp.maximum(gcnt, 1.0)
    logits = pooled[:, 0:C]
    m = jnp.max(logits, axis=1, keepdims=True)
    z = logits - m
    out_ref[...] = z - jnp.log(jnp.sum(jnp.exp(z), axis=1, keepdims=True))

  return pl.pallas_call(
      body,
      out_shape=jax.ShapeDtypeStruct((G, C), jnp.float32),
  )(acc, r2, b2, batch2d)


def kernel(x, edge_index, edge_attr, batch,
           lin1_w, lin1_b, root1_w, bias1,
           lin2_w, lin2_b, root2_w, bias2):
  f32 = jnp.float32
  src = edge_index[0].astype(jnp.int32)
  dst = edge_index[1].astype(jnp.int32)
  ea = edge_attr.reshape(E).astype(f32)

  npad = E_PAD - E
  src2d = jnp.concatenate([src, jnp.zeros((npad,), jnp.int32)]).reshape(
      N_CHUNKS, CHUNK)
  dst2d = jnp.concatenate(
      [dst, jnp.full((npad,), DUMMY_DST, jnp.int32)]).reshape(N_CHUNKS, CHUNK)
  ea_pad = jnp.concatenate([ea, jnp.zeros((npad,), f32)])

  w1 = lin1_w.reshape(DF, H)
  b1m = lin1_b.reshape(DF, H)
  w1_pq = jnp.concatenate([w1, b1m], axis=1)
  bias1r = bias1.reshape(1, H)

  w2 = lin2_w.reshape(H, C)
  b2m = lin2_b.reshape(H, C)
  zpad = jnp.zeros((H, H - C), f32)
  w2_pq = jnp.concatenate([w2, zpad, b2m, zpad], axis=1)
  root2p = jnp.concatenate([root2_w, zpad], axis=1)
  bias2r = jnp.concatenate([bias2, jnp.zeros((H - C,), f32)]).reshape(1, H)

  batch2d = batch.astype(jnp.int32).reshape(N, 1)

  pq1, r1 = _tc_tables_in(x, w1_pq, root1_w)
  acc1 = _sc_aggregate(pq1, src2d, dst2d, ea_pad)
  pq2, r2 = _tc_mid(acc1, r1, bias1r, w2_pq, root2p)
  acc2 = _sc_aggregate(pq2, src2d, dst2d, ea_pad)
  return _tc_pool(acc2, r2, bias2r, batch2d)
